# baseline (speedup 1.0000x reference)
import functools

import jax
import jax.numpy as jnp
from jax import lax
from jax.experimental import pallas as pl
from jax.experimental.pallas import tpu as pltpu
from jax.experimental.pallas import tpu_sc as plsc

N = 256
E = 32
D = 128
H = 8
DH = D // H
SCALE = 1.0 / (DH ** 0.5)
NEG = -1e30

NW = 32
VPW = N // NW
NLANE = 16
NVEC = N // NLANE


def _dott(a, b):
    return lax.dot_general(a, b, (((1,), (1,)), ((), ())),
                           preferred_element_type=jnp.float32,
                           precision=lax.Precision.HIGHEST)


def _tc1_body(x_ref, ei_ref, ea_ref, wlin_ref, wedge_ref, ipw_ref, bqkv_ref,
              w_out, v_out, has_out):
    Mf = (ei_ref[...] != 0).astype(jnp.float32)
    Mt = Mf.T
    Wh = _dott(x_ref[...], wlin_ref[...])
    We = _dott(ea_ref[...], wedge_ref[...])
    ipw = ipw_ref[...]
    Wq, Wk, Wv = ipw[0:D], ipw[D:2 * D], ipw[2 * D:3 * D]
    bq = bqkv_ref[0:1, :]
    bk = bqkv_ref[1:2, :]
    bv = bqkv_ref[2:3, :]
    Q = _dott(Wh, Wq) + bq
    Kx = _dott(Wh, Wk)
    V = _dott(Wh, Wv) + bv
    Ke = _dott(We, Wk) + bk

    U = _dott(Mt, Mt)
    has_out[...] = (jnp.sum(Mt, axis=1, keepdims=True) > 0.0).astype(
        jnp.float32)

    for h in range(H):
        sl = slice(h * DH, (h + 1) * DH)
        osl = slice(h * N, (h + 1) * N)
        Qh, Kxh, Vh, Keh = Q[:, sl], Kx[:, sl], V[:, sl], Ke[:, sl]
        Ah = _dott(Qh, Kxh) * SCALE
        mA = jnp.max(jnp.where(U > 0, Ah, NEG), axis=1, keepdims=True)
        P = jnp.where(U > 0, jnp.exp(Ah - mA), 0.0)
        Bh = _dott(Qh, Keh) * SCALE
        mB = jnp.max(jnp.where(Mt > 0, Bh, NEG), axis=1, keepdims=True)
        s = jnp.where(Mt > 0, jnp.exp(Bh - mB), 0.0)
        w_out[:, osl] = P * _dot_nt(s, Mf)
    v_out[...] = V.T


def _dot_nt(a, b):
    return lax.dot_general(a, b, (((1,), (0,)), ((), ())),
                           preferred_element_type=jnp.float32,
                           precision=lax.Precision.HIGHEST)


def _sc_body(w_hbm, v_hbm, out_hbm, w_v, v_v, o_v, sem_w, sem_v):
    wid = lax.axis_index("s") * 2 + lax.axis_index("c")
    base = wid * VPW
    cp_w = pltpu.async_copy(w_hbm.at[pl.ds(base, VPW)], w_v, sem_w)
    cp_v = pltpu.async_copy(v_hbm, v_v, sem_v)
    cp_w.wait()
    cp_v.wait()
    lanes = lax.iota(jnp.int32, NLANE)
    bfly = [lanes ^ k for k in (8, 4, 2, 1)]
    masks = [(lanes & k) == 0 for k in (1, 2, 4, 8)]
    zero = jnp.zeros((NLANE,), jnp.float32)

    def _perm(vec, idx):
        return vec.at[idx].get(mode="promise_in_bounds")

    def _allsum(vec):
        for idx in bfly:
            vec = vec + _perm(vec, idx)
        return vec

    def _tree(vals, op):
        while len(vals) > 1:
            vals = [op(vals[i], vals[i + 1]) if i + 1 < len(vals) else vals[i]
                    for i in range(0, len(vals), 2)]
        return vals[0]

    def _tsum(vs):
        for k in range(4):
            idx = bfly[3 - k]
            nxt = []
            for i in range(0, len(vs), 2):
                aa, bb = vs[i], vs[i + 1]
                pa, pb = _perm(aa, idx), _perm(bb, idx)
                nxt.append(jnp.where(masks[k], aa, pb) +
                           jnp.where(masks[k], pa, bb))
            vs = nxt
        return vs[0]

    def h_loop(h, vi):
        col = h * N
        drow = h * DH

        def j_loop(j, carry):
            z = carry[0]
            accd = list(carry[1:])
            wj = w_v[vi, pl.ds(col + NLANE * j, NLANE)]
            z = z + wj
            jb = NLANE * j
            for d in range(DH):
                accd[d] = accd[d] + wj * v_v[drow + d, pl.ds(jb, NLANE)]
            return (z, *accd)

        z, *accd = lax.fori_loop(0, NVEC, j_loop, (zero,) * 17)
        rz = 1.0 / _allsum(z)
        o_v[vi, pl.ds(drow, DH)] = _tsum(accd) * rz
        return vi

    def v_loop(vi, carry):
        lax.fori_loop(0, H, h_loop, vi)
        return carry

    lax.fori_loop(0, VPW, v_loop, 0)
    pltpu.sync_copy(o_v, out_hbm.at[pl.ds(base, VPW)])


def _tc2_body(attn_ref, has_ref, wout_ref, bout_ref, out_ref):
    o = _dott(attn_ref[...], wout_ref[...]) + bout_ref[...]
    o = jnp.where(has_ref[...] > 0.0, o, 0.0)
    out_ref[...] = jnp.maximum(o, 0.0)


@jax.jit
def kernel(x, edge_index, edge_attr, W_lin, W_edge, in_proj_w, in_proj_b,
           out_proj_w, out_proj_b):
    bqkv = in_proj_b.reshape(3, D)

    tc1 = pl.pallas_call(
        _tc1_body,
        out_shape=(
            jax.ShapeDtypeStruct((N, H * N), jnp.float32),
            jax.ShapeDtypeStruct((D, N), jnp.float32),
            jax.ShapeDtypeStruct((N, 1), jnp.float32),
        ),
    )
    w_l, v_full, hasf = tc1(x, edge_index.astype(jnp.int32), edge_attr,
                            W_lin, W_edge, in_proj_w, bqkv)

    sc = functools.partial(
        pl.kernel,
        out_type=jax.ShapeDtypeStruct((N, D), jnp.float32),
        scratch_types=[
            pltpu.VMEM((VPW, H * N), jnp.float32),
            pltpu.VMEM((D, N), jnp.float32),
            pltpu.VMEM((VPW, D), jnp.float32),
            pltpu.SemaphoreType.DMA,
            pltpu.SemaphoreType.DMA,
        ],
        mesh=plsc.VectorSubcoreMesh(core_axis_name="c", subcore_axis_name="s"),
    )(_sc_body)
    attn = sc(w_l, v_full)

    tc2 = pl.pallas_call(
        _tc2_body,
        out_shape=jax.ShapeDtypeStruct((N, D), jnp.float32),
    )
    return tc2(attn, hasf, out_proj_w, out_proj_b.reshape(1, D))


# speedup vs baseline: 1.0094x; 1.0094x over previous
import functools

import jax
import jax.numpy as jnp
from jax import lax
from jax.experimental import pallas as pl
from jax.experimental.pallas import tpu as pltpu
from jax.experimental.pallas import tpu_sc as plsc

N = 256
E = 32
D = 128
H = 8
DH = D // H
SCALE = 1.0 / (DH ** 0.5)
NEG = -1e30

NW = 32
VPW = N // NW
NLANE = 16
NVEC = N // NLANE


def _dott(a, b):
    return lax.dot_general(a, b, (((1,), (1,)), ((), ())),
                           preferred_element_type=jnp.float32,
                           precision=lax.Precision.HIGHEST)


def _tc1_body(x_ref, ei_ref, ea_ref, wlin_ref, wedge_ref, ipw_ref, bqkv_ref,
              w_out, v_out, has_out):
    Mf = (ei_ref[...] != 0).astype(jnp.float32)
    Mt = Mf.T
    Wh = _dott(x_ref[...], wlin_ref[...])
    We = _dott(ea_ref[...], wedge_ref[...])
    ipw = ipw_ref[...]
    Wq, Wk, Wv = ipw[0:D], ipw[D:2 * D], ipw[2 * D:3 * D]
    bq = bqkv_ref[0:1, :]
    bk = bqkv_ref[1:2, :]
    bv = bqkv_ref[2:3, :]
    Q = _dott(Wh, Wq) + bq
    Kx = _dott(Wh, Wk)
    V = _dott(Wh, Wv) + bv
    Ke = _dott(We, Wk) + bk

    U = _dott(Mt, Mt)
    has_out[...] = (jnp.sum(Mt, axis=1, keepdims=True) > 0.0).astype(
        jnp.float32)

    for h in range(H):
        sl = slice(h * DH, (h + 1) * DH)
        osl = slice(h * N, (h + 1) * N)
        Qh, Kxh, Vh, Keh = Q[:, sl], Kx[:, sl], V[:, sl], Ke[:, sl]
        Ah = _dott(Qh, Kxh) * SCALE
        mA = jnp.max(jnp.where(U > 0, Ah, NEG), axis=1, keepdims=True)
        P = jnp.where(U > 0, jnp.exp(Ah - mA), 0.0)
        Bh = _dott(Qh, Keh) * SCALE
        mB = jnp.max(jnp.where(Mt > 0, Bh, NEG), axis=1, keepdims=True)
        s = jnp.where(Mt > 0, jnp.exp(Bh - mB), 0.0)
        w_out[:, osl] = P * _dot_nt(s, Mf)
    v_out[...] = V


def _dot_nt(a, b):
    return lax.dot_general(a, b, (((1,), (0,)), ((), ())),
                           preferred_element_type=jnp.float32,
                           precision=lax.Precision.HIGHEST)


def _sc_body(w_hbm, v_hbm, out_hbm, w_v, v_v, o_v, sem_w, sem_v):
    wid = lax.axis_index("s") * 2 + lax.axis_index("c")
    base = wid * VPW
    cp_w = pltpu.async_copy(w_hbm.at[pl.ds(base, VPW)], w_v, sem_w)
    cp_v = pltpu.async_copy(v_hbm, v_v, sem_v)
    cp_w.wait()
    cp_v.wait()
    lanes = lax.iota(jnp.int32, NLANE)
    bfly = [lanes ^ k for k in (8, 4, 2, 1)]
    lane_idx = [jnp.full((NLANE,), t, dtype=jnp.int32) for t in range(NLANE)]
    zero = jnp.zeros((NLANE,), jnp.float32)

    def _perm(vec, idx):
        return vec.at[idx].get(mode="promise_in_bounds")

    def _allsum(vec):
        for idx in bfly:
            vec = vec + _perm(vec, idx)
        return vec

    def _tree(vals, op):
        while len(vals) > 1:
            vals = [op(vals[i], vals[i + 1]) if i + 1 < len(vals) else vals[i]
                    for i in range(0, len(vals), 2)]
        return vals[0]

    def h_loop(h, vi):
        col = h * N
        drow = h * DH

        def j_loop(j, carry):
            z = carry[0]
            accs = list(carry[1:])
            wj = w_v[vi, pl.ds(col + NLANE * j, NLANE)]
            z = z + wj
            jb = NLANE * j
            for t in range(NLANE):
                wb = _perm(wj, lane_idx[t])
                row = v_v[jb + t, pl.ds(drow, DH)]
                k = t % 8
                accs[k] = accs[k] + wb * row
            return (z, *accs)

        z, *accs = lax.fori_loop(0, NVEC, j_loop, (zero,) * 9)
        rz = 1.0 / _allsum(z)
        o_v[vi, pl.ds(drow, DH)] = _tree(accs, jnp.add) * rz
        return vi

    def v_loop(vi, carry):
        lax.fori_loop(0, H, h_loop, vi)
        return carry

    lax.fori_loop(0, VPW, v_loop, 0)
    pltpu.sync_copy(o_v, out_hbm.at[pl.ds(base, VPW)])


def _tc2_body(attn_ref, has_ref, wout_ref, bout_ref, out_ref):
    o = _dott(attn_ref[...], wout_ref[...]) + bout_ref[...]
    o = jnp.where(has_ref[...] > 0.0, o, 0.0)
    out_ref[...] = jnp.maximum(o, 0.0)


@jax.jit
def kernel(x, edge_index, edge_attr, W_lin, W_edge, in_proj_w, in_proj_b,
           out_proj_w, out_proj_b):
    bqkv = in_proj_b.reshape(3, D)

    tc1 = pl.pallas_call(
        _tc1_body,
        out_shape=(
            jax.ShapeDtypeStruct((N, H * N), jnp.float32),
            jax.ShapeDtypeStruct((N, D), jnp.float32),
            jax.ShapeDtypeStruct((N, 1), jnp.float32),
        ),
    )
    w_l, v_full, hasf = tc1(x, edge_index, edge_attr,
                            W_lin, W_edge, in_proj_w, bqkv)

    sc = functools.partial(
        pl.kernel,
        out_type=jax.ShapeDtypeStruct((N, D), jnp.float32),
        scratch_types=[
            pltpu.VMEM((VPW, H * N), jnp.float32),
            pltpu.VMEM((N, D), jnp.float32),
            pltpu.VMEM((VPW, D), jnp.float32),
            pltpu.SemaphoreType.DMA,
            pltpu.SemaphoreType.DMA,
        ],
        mesh=plsc.VectorSubcoreMesh(core_axis_name="c", subcore_axis_name="s"),
    )(_sc_body)
    attn = sc(w_l, v_full)

    tc2 = pl.pallas_call(
        _tc2_body,
        out_shape=jax.ShapeDtypeStruct((N, D), jnp.float32),
    )
    return tc2(attn, hasf, out_proj_w, out_proj_b.reshape(1, D))


# speedup vs baseline: 1.0643x; 1.0543x over previous
import functools

import jax
import jax.numpy as jnp
from jax import lax
from jax.experimental import pallas as pl
from jax.experimental.pallas import tpu as pltpu
from jax.experimental.pallas import tpu_sc as plsc

N = 256
E = 32
D = 128
H = 8
DH = D // H
SCALE = 1.0 / (DH ** 0.5)
NEG = -1e30

NW = 32
VPW = N // NW
NLANE = 16
NVEC = N // NLANE


def _dott(a, b):
    return lax.dot_general(a, b, (((1,), (1,)), ((), ())),
                           preferred_element_type=jnp.float32,
                           precision=lax.Precision.HIGHEST)


def _tc1_body(x_ref, ei_ref, ea_t_ref, wlin_ref, wedge_t_ref, ipw_ref,
              bqkv_ref, w_out, v_out, has_out):
    Mf = (ei_ref[...] != 0).astype(jnp.float32)
    Mt = Mf.T
    Wh = _dott(x_ref[...], wlin_ref[...])
    We = lax.dot_general(ea_t_ref[...], wedge_t_ref[...],
                         (((0,), (0,)), ((), ())),
                         preferred_element_type=jnp.float32,
                         precision=lax.Precision.HIGHEST)
    ipw = ipw_ref[...]
    Wq, Wk, Wv = ipw[0:D], ipw[D:2 * D], ipw[2 * D:3 * D]
    bq = bqkv_ref[0:1, :]
    bk = bqkv_ref[1:2, :]
    bv = bqkv_ref[2:3, :]
    Q = _dott(Wh, Wq) + bq
    Kx = _dott(Wh, Wk)
    V = _dott(Wh, Wv) + bv
    Ke = _dott(We, Wk) + bk

    U = _dott(Mt, Mt)
    has_out[...] = (jnp.sum(Mt, axis=1, keepdims=True) > 0.0).astype(
        jnp.float32)

    for h in range(H):
        sl = slice(h * DH, (h + 1) * DH)
        osl = slice(h * N, (h + 1) * N)
        Qh, Kxh, Vh, Keh = Q[:, sl], Kx[:, sl], V[:, sl], Ke[:, sl]
        Ah = _dott(Qh, Kxh) * SCALE
        mA = jnp.max(jnp.where(U > 0, Ah, NEG), axis=1, keepdims=True)
        P = jnp.where(U > 0, jnp.exp(Ah - mA), 0.0)
        Bh = _dott(Qh, Keh) * SCALE
        mB = jnp.max(jnp.where(Mt > 0, Bh, NEG), axis=1, keepdims=True)
        s = jnp.where(Mt > 0, jnp.exp(Bh - mB), 0.0)
        w_out[:, osl] = P * _dot_nt(s, Mf)
    v_out[...] = V


def _dot_nt(a, b):
    return lax.dot_general(a, b, (((1,), (0,)), ((), ())),
                           preferred_element_type=jnp.float32,
                           precision=lax.Precision.HIGHEST)


def _sc_body(w_hbm, v_hbm, out_hbm, w_v, v_v, o_v, sem_w, sem_v):
    wid = lax.axis_index("s") * 2 + lax.axis_index("c")
    base = wid * VPW
    cp_w = pltpu.async_copy(w_hbm.at[pl.ds(base, VPW)], w_v, sem_w)
    cp_v = pltpu.async_copy(v_hbm, v_v, sem_v)
    cp_w.wait()
    cp_v.wait()
    lanes = lax.iota(jnp.int32, NLANE)
    bfly = [lanes ^ k for k in (8, 4, 2, 1)]
    lane_idx = [jnp.full((NLANE,), t, dtype=jnp.int32) for t in range(NLANE)]
    zero = jnp.zeros((NLANE,), jnp.float32)

    def _perm(vec, idx):
        return vec.at[idx].get(mode="promise_in_bounds")

    def _allsum(vec):
        for idx in bfly:
            vec = vec + _perm(vec, idx)
        return vec

    def _tree(vals, op):
        while len(vals) > 1:
            vals = [op(vals[i], vals[i + 1]) if i + 1 < len(vals) else vals[i]
                    for i in range(0, len(vals), 2)]
        return vals[0]

    def h_loop(h, vi):
        col = h * N
        drow = h * DH

        def j_loop(j, carry):
            z = carry[0]
            accs = list(carry[1:])
            wj = w_v[vi, pl.ds(col + NLANE * j, NLANE)]
            z = z + wj
            jb = NLANE * j
            for t in range(NLANE):
                wb = _perm(wj, lane_idx[t])
                row = v_v[jb + t, pl.ds(drow, DH)]
                k = t % 8
                accs[k] = accs[k] + wb * row
            return (z, *accs)

        z, *accs = lax.fori_loop(0, NVEC, j_loop, (zero,) * 9)
        rz = 1.0 / _allsum(z)
        o_v[vi, pl.ds(drow, DH)] = _tree(accs, jnp.add) * rz
        return vi

    def v_loop(vi, carry):
        lax.fori_loop(0, H, h_loop, vi)
        return carry

    lax.fori_loop(0, VPW, v_loop, 0)
    pltpu.sync_copy(o_v, out_hbm.at[pl.ds(base, VPW)])


def _tc2_body(attn_ref, has_ref, wout_ref, bout_ref, out_ref):
    o = _dott(attn_ref[...], wout_ref[...]) + bout_ref[...]
    o = jnp.where(has_ref[...] > 0.0, o, 0.0)
    out_ref[...] = jnp.maximum(o, 0.0)


@jax.jit
def kernel(x, edge_index, edge_attr, W_lin, W_edge, in_proj_w, in_proj_b,
           out_proj_w, out_proj_b):
    bqkv = in_proj_b.reshape(3, D)

    tc1 = pl.pallas_call(
        _tc1_body,
        out_shape=(
            jax.ShapeDtypeStruct((N, H * N), jnp.float32),
            jax.ShapeDtypeStruct((N, D), jnp.float32),
            jax.ShapeDtypeStruct((N, 1), jnp.float32),
        ),
    )
    w_l, v_full, hasf = tc1(x, edge_index, edge_attr.T,
                            W_lin, W_edge.T, in_proj_w, bqkv)

    sc = functools.partial(
        pl.kernel,
        out_type=jax.ShapeDtypeStruct((N, D), jnp.float32),
        scratch_types=[
            pltpu.VMEM((VPW, H * N), jnp.float32),
            pltpu.VMEM((N, D), jnp.float32),
            pltpu.VMEM((VPW, D), jnp.float32),
            pltpu.SemaphoreType.DMA,
            pltpu.SemaphoreType.DMA,
        ],
        mesh=plsc.VectorSubcoreMesh(core_axis_name="c", subcore_axis_name="s"),
    )(_sc_body)
    attn = sc(w_l, v_full)

    tc2 = pl.pallas_call(
        _tc2_body,
        out_shape=jax.ShapeDtypeStruct((N, D), jnp.float32),
    )
    return tc2(attn, hasf, out_proj_w, out_proj_b.reshape(1, D))


# speedup vs baseline: 1.1983x; 1.1259x over previous
import functools

import jax
import jax.numpy as jnp
from jax import lax
from jax.experimental import pallas as pl
from jax.experimental.pallas import tpu as pltpu
from jax.experimental.pallas import tpu_sc as plsc

N = 256
E = 32
D = 128
H = 8
DH = D // H
SCALE = 1.0 / (DH ** 0.5)
NEG = -1e30

NW = 32
VPW = N // NW
NLANE = 16
NVEC = N // NLANE


def _dott(a, b):
    return lax.dot_general(a, b, (((1,), (1,)), ((), ())),
                           preferred_element_type=jnp.float32,
                           precision=lax.Precision.DEFAULT)


def _tc1_body(x_ref, ei_ref, ea_t_ref, wlin_ref, wedge_t_ref, ipw_ref,
              bqkv_ref, w_out, v_out, has_out):
    Mf = (ei_ref[...] != 0).astype(jnp.float32)
    Mt = Mf.T
    Wh = _dott(x_ref[...], wlin_ref[...])
    We = lax.dot_general(ea_t_ref[...], wedge_t_ref[...],
                         (((0,), (0,)), ((), ())),
                         preferred_element_type=jnp.float32,
                         precision=lax.Precision.DEFAULT)
    ipw = ipw_ref[...]
    Wq, Wk, Wv = ipw[0:D], ipw[D:2 * D], ipw[2 * D:3 * D]
    bq = bqkv_ref[0:1, :]
    bk = bqkv_ref[1:2, :]
    bv = bqkv_ref[2:3, :]
    Q = _dott(Wh, Wq) + bq
    Kx = _dott(Wh, Wk)
    V = _dott(Wh, Wv) + bv
    Ke = _dott(We, Wk) + bk

    U = _dott(Mt, Mt)
    has_out[...] = (jnp.sum(Mt, axis=1, keepdims=True) > 0.0).astype(
        jnp.float32)

    for h in range(H):
        sl = slice(h * DH, (h + 1) * DH)
        osl = slice(h * N, (h + 1) * N)
        Qh, Kxh, Vh, Keh = Q[:, sl], Kx[:, sl], V[:, sl], Ke[:, sl]
        Ah = _dott(Qh, Kxh) * SCALE
        mA = jnp.max(jnp.where(U > 0, Ah, NEG), axis=1, keepdims=True)
        P = jnp.where(U > 0, jnp.exp(Ah - mA), 0.0)
        Bh = _dott(Qh, Keh) * SCALE
        mB = jnp.max(jnp.where(Mt > 0, Bh, NEG), axis=1, keepdims=True)
        s = jnp.where(Mt > 0, jnp.exp(Bh - mB), 0.0)
        w_out[:, osl] = P * _dot_nt(s, Mf)
    v_out[...] = V


def _dot_nt(a, b):
    return lax.dot_general(a, b, (((1,), (0,)), ((), ())),
                           preferred_element_type=jnp.float32,
                           precision=lax.Precision.DEFAULT)


def _sc_body(w_hbm, v_hbm, out_hbm, w_v, v_v, o_v, sem_w, sem_v):
    wid = lax.axis_index("s") * 2 + lax.axis_index("c")
    base = wid * VPW
    cp_w = pltpu.async_copy(w_hbm.at[pl.ds(base, VPW)], w_v, sem_w)
    cp_v = pltpu.async_copy(v_hbm, v_v, sem_v)
    cp_w.wait()
    cp_v.wait()
    lanes = lax.iota(jnp.int32, NLANE)
    bfly = [lanes ^ k for k in (8, 4, 2, 1)]
    lane_idx = [jnp.full((NLANE,), t, dtype=jnp.int32) for t in range(NLANE)]
    zero = jnp.zeros((NLANE,), jnp.float32)

    def _perm(vec, idx):
        return vec.at[idx].get(mode="promise_in_bounds")

    def _allsum(vec):
        for idx in bfly:
            vec = vec + _perm(vec, idx)
        return vec

    def _tree(vals, op):
        while len(vals) > 1:
            vals = [op(vals[i], vals[i + 1]) if i + 1 < len(vals) else vals[i]
                    for i in range(0, len(vals), 2)]
        return vals[0]

    def h_loop(h, vi):
        col = h * N
        drow = h * DH

        def j_loop(j, carry):
            z = carry[0]
            accs = list(carry[1:])
            wj = w_v[vi, pl.ds(col + NLANE * j, NLANE)]
            z = z + wj
            jb = NLANE * j
            for t in range(NLANE):
                wb = _perm(wj, lane_idx[t])
                row = v_v[jb + t, pl.ds(drow, DH)]
                k = t % 8
                accs[k] = accs[k] + wb * row
            return (z, *accs)

        z, *accs = lax.fori_loop(0, NVEC, j_loop, (zero,) * 9)
        rz = 1.0 / _allsum(z)
        o_v[vi, pl.ds(drow, DH)] = _tree(accs, jnp.add) * rz
        return vi

    def v_loop(vi, carry):
        lax.fori_loop(0, H, h_loop, vi)
        return carry

    lax.fori_loop(0, VPW, v_loop, 0)
    pltpu.sync_copy(o_v, out_hbm.at[pl.ds(base, VPW)])


def _tc2_body(attn_ref, has_ref, wout_ref, bout_ref, out_ref):
    o = _dott(attn_ref[...], wout_ref[...]) + bout_ref[...]
    o = jnp.where(has_ref[...] > 0.0, o, 0.0)
    out_ref[...] = jnp.maximum(o, 0.0)


@jax.jit
def kernel(x, edge_index, edge_attr, W_lin, W_edge, in_proj_w, in_proj_b,
           out_proj_w, out_proj_b):
    bqkv = in_proj_b.reshape(3, D)

    tc1 = pl.pallas_call(
        _tc1_body,
        out_shape=(
            jax.ShapeDtypeStruct((N, H * N), jnp.float32),
            jax.ShapeDtypeStruct((N, D), jnp.float32),
            jax.ShapeDtypeStruct((N, 1), jnp.float32),
        ),
    )
    w_l, v_full, hasf = tc1(x, edge_index, edge_attr.T,
                            W_lin, W_edge.T, in_proj_w, bqkv)

    sc = functools.partial(
        pl.kernel,
        out_type=jax.ShapeDtypeStruct((N, D), jnp.float32),
        scratch_types=[
            pltpu.VMEM((VPW, H * N), jnp.float32),
            pltpu.VMEM((N, D), jnp.float32),
            pltpu.VMEM((VPW, D), jnp.float32),
            pltpu.SemaphoreType.DMA,
            pltpu.SemaphoreType.DMA,
        ],
        mesh=plsc.VectorSubcoreMesh(core_axis_name="c", subcore_axis_name="s"),
    )(_sc_body)
    attn = sc(w_l, v_full)

    tc2 = pl.pallas_call(
        _tc2_body,
        out_shape=jax.ShapeDtypeStruct((N, D), jnp.float32),
    )
    return tc2(attn, hasf, out_proj_w, out_proj_b.reshape(1, D))
